# Initial kernel scaffold; baseline (speedup 1.0000x reference)
import functools

import jax
import jax.numpy as jnp
from jax import lax
from jax.experimental import pallas as pl
from jax.experimental.pallas import tpu as pltpu
from jax.experimental.pallas import tpu_sc as plsc

D_MODEL = 32
LANES = 128
NW = 32
NSUB = 8


def _make_kernel(n_rows: int):
    rows_per_w = n_rows // NW
    n_chunks = rows_per_w // NSUB
    mesh = plsc.VectorSubcoreMesh(core_axis_name="c", subcore_axis_name="s")

    @functools.partial(
        pl.kernel,
        out_type=jax.ShapeDtypeStruct((n_rows, LANES, D_MODEL), jnp.float32),
        mesh=mesh,
        scratch_types=[
            pltpu.VMEM((NSUB, LANES), jnp.int32),
            pltpu.VMEM((NSUB, LANES, D_MODEL), jnp.float32),
            pltpu.SemaphoreType.DMA,
        ],
    )
    def gather_kernel(idx_hbm, tab_hbm, out_hbm, idx_v, rows_v, gsem):
        wid = lax.axis_index("s") * 2 + lax.axis_index("c")
        row0 = wid * rows_per_w

        def chunk(g, carry):
            r0 = row0 + g * NSUB
            pltpu.sync_copy(idx_hbm.at[pl.ds(r0, NSUB)], idx_v)
            copies = [
                pltpu.async_copy(tab_hbm.at[idx_v.at[j]], rows_v.at[j], gsem)
                for j in range(NSUB)
            ]
            for c in copies:
                c.wait()
            pltpu.sync_copy(rows_v, out_hbm.at[pl.ds(r0, NSUB)])
            return carry

        lax.fori_loop(0, n_chunks, chunk, 0)

    return gather_kernel


@jax.jit
def kernel(stem_idx, embedding_weight):
    b, s = stem_idx.shape
    n = b * s
    idx2d = stem_idx.astype(jnp.int32).reshape(n // LANES, LANES)
    out = _make_kernel(n // LANES)(idx2d, embedding_weight)
    return out.reshape(b, s, D_MODEL)


# speedup vs baseline: 4.8100x; 4.8100x over previous
import functools

import jax
import jax.numpy as jnp
from jax import lax
from jax.experimental import pallas as pl
from jax.experimental.pallas import tpu as pltpu
from jax.experimental.pallas import tpu_sc as plsc

D_MODEL = 32
LANES = 128
NW = 32
NSUB = 8


def _make_kernel(n_rows: int):
    rows_per_w = n_rows // NW
    n_chunks = rows_per_w // NSUB
    mesh = plsc.VectorSubcoreMesh(core_axis_name="c", subcore_axis_name="s")

    @functools.partial(
        pl.kernel,
        out_type=jax.ShapeDtypeStruct((n_rows, LANES, D_MODEL), jnp.float32),
        mesh=mesh,
        scratch_types=[
            pltpu.VMEM((NSUB, LANES), jnp.int32),
            pltpu.VMEM((NSUB, LANES, D_MODEL), jnp.float32),
            pltpu.SemaphoreType.DMA,
        ],
        compiler_params=pltpu.CompilerParams(use_tc_tiling_on_sc=False),
    )
    def gather_kernel(idx_hbm, tab_hbm, out_hbm, idx_v, rows_v, gsem):
        wid = lax.axis_index("s") * 2 + lax.axis_index("c")
        row0 = wid * rows_per_w

        def chunk(g, carry):
            r0 = row0 + g * NSUB
            pltpu.sync_copy(idx_hbm.at[pl.ds(r0, NSUB)], idx_v)
            copies = [
                pltpu.async_copy(tab_hbm.at[idx_v.at[j]], rows_v.at[j], gsem)
                for j in range(NSUB)
            ]
            for c in copies:
                c.wait()
            pltpu.sync_copy(rows_v, out_hbm.at[pl.ds(r0, NSUB)])
            return carry

        lax.fori_loop(0, n_chunks, chunk, 0)

    return gather_kernel


@jax.jit
def kernel(stem_idx, embedding_weight):
    b, s = stem_idx.shape
    n = b * s
    idx2d = stem_idx.astype(jnp.int32).reshape(n // LANES, LANES)
    out = _make_kernel(n // LANES)(idx2d, embedding_weight)
    return out.reshape(b, s, D_MODEL)


# speedup vs baseline: 5.0472x; 1.0493x over previous
import functools

import jax
import jax.numpy as jnp
from jax import lax
from jax.experimental import pallas as pl
from jax.experimental.pallas import tpu as pltpu
from jax.experimental.pallas import tpu_sc as plsc

D_MODEL = 32
LANES = 128
NW = 32
NSUB = 4
NBUF = 4
LOOKAHEAD = 2


def _make_kernel(n_rows: int):
    rows_per_w = n_rows // NW
    n_chunks = rows_per_w // NSUB
    assert n_chunks % NBUF == 0 and n_chunks > NBUF
    mesh = plsc.VectorSubcoreMesh(core_axis_name="c", subcore_axis_name="s")

    @functools.partial(
        pl.kernel,
        out_type=jax.ShapeDtypeStruct((n_rows, LANES, D_MODEL), jnp.float32),
        mesh=mesh,
        scratch_types=(
            [pltpu.VMEM((NSUB, LANES), jnp.int32) for _ in range(NBUF)]
            + [pltpu.VMEM((NSUB, LANES, D_MODEL), jnp.float32) for _ in range(NBUF)]
            + [pltpu.SemaphoreType.DMA for _ in range(2 * NBUF)]
        ),
        compiler_params=pltpu.CompilerParams(use_tc_tiling_on_sc=False),
    )
    def gather_kernel(idx_hbm, tab_hbm, out_hbm, *scratch):
        idxs = scratch[:NBUF]
        rows = scratch[NBUF:2 * NBUF]
        gsems = scratch[2 * NBUF:3 * NBUF]
        osems = scratch[3 * NBUF:]

        wid = lax.axis_index("s") * 2 + lax.axis_index("c")
        row0 = wid * rows_per_w

        def fire(g, b):
            r0 = row0 + g * NSUB
            pltpu.sync_copy(idx_hbm.at[pl.ds(r0, NSUB)], idxs[b])
            for j in range(NSUB):
                pltpu.async_copy(tab_hbm.at[idxs[b].at[j]], rows[b].at[j], gsems[b])

        def wait_gathers(b):
            pltpu.make_async_copy(out_hbm.at[pl.ds(0, NSUB)], rows[b], gsems[b]).wait()

        def wait_writeout(b):
            pltpu.make_async_copy(out_hbm.at[pl.ds(0, NSUB)], rows[b], osems[b]).wait()

        for g in range(LOOKAHEAD):
            fire(g, g % NBUF)

        @pl.loop(0, n_chunks, step=NBUF)
        def body(t):
            for b in range(NBUF):
                g = t + b
                bn = (b + LOOKAHEAD) % NBUF
                nxt = g + LOOKAHEAD

                @pl.when(nxt < n_chunks)
                def _():
                    @pl.when(nxt >= NBUF)
                    def _():
                        wait_writeout(bn)
                    fire(nxt, bn)

                wait_gathers(b)
                pltpu.async_copy(
                    rows[b], out_hbm.at[pl.ds(row0 + g * NSUB, NSUB)], osems[b]
                )

        for b in range(NBUF):
            wait_writeout(b)

    return gather_kernel


@jax.jit
def kernel(stem_idx, embedding_weight):
    b, s = stem_idx.shape
    n = b * s
    idx2d = stem_idx.astype(jnp.int32).reshape(n // LANES, LANES)
    out = _make_kernel(n // LANES)(idx2d, embedding_weight)
    return out.reshape(b, s, D_MODEL)
